# Initial kernel scaffold; baseline (speedup 1.0000x reference)
import jax
import jax.numpy as jnp
from jax.experimental import pallas as pl


def kernel(x, edge_index, W_self1, W_neigh1, b1, W_self2, W_neigh2, b2):
    raise NotImplementedError("write your pallas kernel here")


# speedup vs baseline: 5.5854x; 5.5854x over previous
import functools

import jax
import jax.numpy as jnp
from jax import lax
from jax.experimental import pallas as pl
from jax.experimental.pallas import tpu as pltpu
from jax.experimental.pallas import tpu_sc as plsc

N = 10000
D = 128
H = 128
E = 320000

NC = 2
NS = 16
NW = NC * NS
CH = 128
NCHUNK = E // CH
NFULL = NCHUNK // NW
NEXTRA = NCHUNK % NW
NPAD = 10112
RPT = NPAD // NS

_mesh = plsc.VectorSubcoreMesh(core_axis_name="c", subcore_axis_name="s")


def _worker_chunk_loop(do_chunk, w):
    def loop_body(k, carry):
        do_chunk(w + k * NW)
        return carry

    lax.fori_loop(0, NFULL, loop_body, 0)

    @pl.when(w < NEXTRA)
    def _():
        do_chunk(NFULL * NW + w)


@functools.partial(
    pl.kernel,
    out_type=jax.ShapeDtypeStruct((NC, NPAD, D), jnp.float32),
    mesh=_mesh,
    scratch_types=[
        pltpu.VMEM((CH,), jnp.int32),
        pltpu.VMEM((CH,), jnp.int32),
        pltpu.VMEM((CH, D), jnp.float32),
        pltpu.VMEM_SHARED((NPAD, D), jnp.float32),
        pltpu.SemaphoreType.DMA,
    ],
)
def _seg(p_hbm, src_hbm, dst_hbm, zrow_hbm, acc_out,
         idx_s, idx_d, rows_v, acc_sh, sem):
    cid = lax.axis_index("c")
    sid = lax.axis_index("s")
    w = sid * NC + cid

    r0 = sid * RPT
    pltpu.sync_copy(zrow_hbm.at[pl.ds(r0, RPT)], acc_sh.at[pl.ds(r0, RPT)])
    plsc.subcore_barrier()

    def do_chunk(j):
        base = j * CH
        pltpu.sync_copy(src_hbm.at[pl.ds(base, CH)], idx_s)
        pltpu.sync_copy(dst_hbm.at[pl.ds(base, CH)], idx_d)
        pltpu.async_copy(p_hbm.at[idx_s], rows_v, sem).wait()
        pltpu.sync_copy(rows_v, acc_sh.at[idx_d], add=True)

    _worker_chunk_loop(do_chunk, w)

    plsc.subcore_barrier()
    pltpu.sync_copy(acc_sh.at[pl.ds(r0, RPT)],
                    acc_out.at[cid, pl.ds(r0, RPT)])


@functools.partial(
    pl.kernel,
    out_type=jax.ShapeDtypeStruct((NC, NPAD, D), jnp.float32),
    mesh=_mesh,
    scratch_types=[
        pltpu.VMEM((CH,), jnp.int32),
        pltpu.VMEM((CH, D), jnp.float32),
        pltpu.VMEM_SHARED((NPAD, D), jnp.float32),
    ],
)
def _deg(dst_hbm, zrow_hbm, ones_hbm, deg_out, idx_d, ones_v, deg_sh):
    cid = lax.axis_index("c")
    sid = lax.axis_index("s")
    w = sid * NC + cid

    r0 = sid * RPT
    pltpu.sync_copy(zrow_hbm.at[pl.ds(r0, RPT)], deg_sh.at[pl.ds(r0, RPT)])
    pltpu.sync_copy(ones_hbm, ones_v)
    plsc.subcore_barrier()

    def do_chunk(j):
        pltpu.sync_copy(dst_hbm.at[pl.ds(j * CH, CH)], idx_d)
        pltpu.sync_copy(ones_v, deg_sh.at[idx_d], add=True)

    _worker_chunk_loop(do_chunk, w)

    plsc.subcore_barrier()
    pltpu.sync_copy(deg_sh.at[pl.ds(r0, RPT)],
                    deg_out.at[cid, pl.ds(r0, RPT)])


def _tc1_body(x_ref, wn_ref, ws_ref, b_ref, p_ref, s_ref):
    h = jnp.log(x_ref[...] + 1.0)
    p_ref[...] = jnp.dot(h, wn_ref[...], preferred_element_type=jnp.float32)
    s_ref[...] = (jnp.dot(h, ws_ref[...], preferred_element_type=jnp.float32)
                  + b_ref[...])


def _tc2_body(s1_ref, a0_ref, a1_ref, d0_ref, d1_ref, wn_ref, ws_ref, b_ref,
              p_ref, s_ref):
    deg = jnp.maximum(d0_ref[...] + d1_ref[...], 1.0)
    h = jax.nn.relu(s1_ref[...] + (a0_ref[...] + a1_ref[...]) / deg)
    nrm = jnp.sqrt(jnp.sum(h * h, axis=-1, keepdims=True))
    h = h / jnp.maximum(nrm, 1e-12)
    p_ref[...] = jnp.dot(h, wn_ref[...], preferred_element_type=jnp.float32)
    s_ref[...] = (jnp.dot(h, ws_ref[...], preferred_element_type=jnp.float32)
                  + b_ref[...])


def _tc3_body(s2_ref, a0_ref, a1_ref, d0_ref, d1_ref, o_ref):
    deg = jnp.maximum(d0_ref[...] + d1_ref[...], 1.0)
    o_ref[...] = s2_ref[...] + (a0_ref[...] + a1_ref[...]) / deg


_tc1 = pl.pallas_call(
    _tc1_body,
    out_shape=[jax.ShapeDtypeStruct((N, H), jnp.float32)] * 2,
)

_tc2 = pl.pallas_call(
    _tc2_body,
    out_shape=[jax.ShapeDtypeStruct((N, H), jnp.float32)] * 2,
)

_tc3 = pl.pallas_call(
    _tc3_body,
    out_shape=jax.ShapeDtypeStruct((N, H), jnp.float32),
)


@jax.jit
def kernel(x, edge_index, W_self1, W_neigh1, b1, W_self2, W_neigh2, b2):
    src = edge_index[0].astype(jnp.int32)
    dst = edge_index[1].astype(jnp.int32)
    zrow = jnp.zeros((NPAD, D), jnp.float32)
    ones = jnp.ones((CH, D), jnp.float32)
    b1r = b1.reshape(1, H)
    b2r = b2.reshape(1, H)

    degp = _deg(dst, zrow, ones)
    p1, s1 = _tc1(x, W_neigh1, W_self1, b1r)
    acc1 = _seg(p1, src, dst, zrow)
    d0 = degp[0, :N]
    d1 = degp[1, :N]
    p2, s2 = _tc2(s1, acc1[0, :N], acc1[1, :N], d0, d1, W_neigh2, W_self2,
                  b2r)
    acc2 = _seg(p2, src, dst, zrow)
    out = _tc3(s2, acc2[0, :N], acc2[1, :N], d0, d1)
    return out


# speedup vs baseline: 9.0744x; 1.6247x over previous
import functools

import jax
import jax.numpy as jnp
from jax import lax
from jax.experimental import pallas as pl
from jax.experimental.pallas import tpu as pltpu
from jax.experimental.pallas import tpu_sc as plsc

N = 10000
D = 128
H = 128
E = 320000

NC = 2
NS = 16
NW = NC * NS
CH = 128
NCHUNK = E // CH
NFULL = NCHUNK // NW
NEXTRA = NCHUNK % NW
NPAD = 10112
RPT = NPAD // NS

_mesh = plsc.VectorSubcoreMesh(core_axis_name="c", subcore_axis_name="s")


def _worker_chunk_loop(do_chunk, w):
    def loop_body(k, carry):
        do_chunk(w + k * NW)
        return carry

    lax.fori_loop(0, NFULL, loop_body, 0)

    @pl.when(w < NEXTRA)
    def _():
        do_chunk(NFULL * NW + w)


@functools.partial(
    pl.kernel,
    out_type=jax.ShapeDtypeStruct((NC, NPAD, D), jnp.float32),
    mesh=_mesh,
    scratch_types=[
        pltpu.VMEM((2, CH), jnp.int32),
        pltpu.VMEM((2, CH), jnp.int32),
        pltpu.VMEM((CH, D), jnp.float32),
        pltpu.VMEM((CH, D), jnp.float32),
        pltpu.VMEM_SHARED((NPAD, D), jnp.float32),
        pltpu.SemaphoreType.DMA,
        pltpu.SemaphoreType.DMA,
    ],
)
def _seg(p_hbm, ei_hbm, zrow_hbm, acc_out,
         idx0, idx1, rows0, rows1, acc_sh, sem0, sem1):
    cid = lax.axis_index("c")
    sid = lax.axis_index("s")
    w = sid * NC + cid

    r0 = sid * RPT
    pltpu.sync_copy(zrow_hbm.at[pl.ds(r0, RPT)], acc_sh.at[pl.ds(r0, RPT)])
    plsc.subcore_barrier()

    cnt = NFULL + jnp.where(w < NEXTRA, 1, 0)

    def jbase(k):
        return jnp.where(k < NFULL, w + k * NW, NFULL * NW + w) * CH

    def fetch(k, idxb, rowsb, semb):
        pltpu.sync_copy(ei_hbm.at[:, pl.ds(jbase(k), CH)], idxb)
        pltpu.make_async_copy(p_hbm.at[idxb.at[0]], rowsb, semb).start()

    def drain(idxb, rowsb, semb):
        pltpu.make_async_copy(p_hbm.at[idxb.at[0]], rowsb, semb).wait()
        pltpu.sync_copy(rowsb, acc_sh.at[idxb.at[1]], add=True)

    fetch(0, idx0, rows0, sem0)

    def pair_body(t, carry):
        k0 = 2 * t
        fetch(k0 + 1, idx1, rows1, sem1)
        drain(idx0, rows0, sem0)

        @pl.when(k0 + 2 < cnt)
        def _():
            fetch(k0 + 2, idx0, rows0, sem0)

        drain(idx1, rows1, sem1)
        return carry

    lax.fori_loop(0, NFULL // 2, pair_body, 0)

    @pl.when(w < NEXTRA)
    def _():
        drain(idx0, rows0, sem0)

    plsc.subcore_barrier()
    pltpu.sync_copy(acc_sh.at[pl.ds(r0, RPT)],
                    acc_out.at[cid, pl.ds(r0, RPT)])


@functools.partial(
    pl.kernel,
    out_type=jax.ShapeDtypeStruct((NC, NPAD, D), jnp.float32),
    mesh=_mesh,
    scratch_types=[
        pltpu.VMEM((CH,), jnp.int32),
        pltpu.VMEM((CH, D), jnp.float32),
        pltpu.VMEM_SHARED((NPAD, D), jnp.float32),
    ],
)
def _deg(ei_hbm, zrow_hbm, ones_hbm, deg_out, idx_d, ones_v, deg_sh):
    cid = lax.axis_index("c")
    sid = lax.axis_index("s")
    w = sid * NC + cid

    r0 = sid * RPT
    pltpu.sync_copy(zrow_hbm.at[pl.ds(r0, RPT)], deg_sh.at[pl.ds(r0, RPT)])
    pltpu.sync_copy(ones_hbm, ones_v)
    plsc.subcore_barrier()

    def do_chunk(j):
        pltpu.sync_copy(ei_hbm.at[1, pl.ds(j * CH, CH)], idx_d)
        pltpu.sync_copy(ones_v, deg_sh.at[idx_d], add=True)

    _worker_chunk_loop(do_chunk, w)

    plsc.subcore_barrier()
    pltpu.sync_copy(deg_sh.at[pl.ds(r0, RPT)],
                    deg_out.at[cid, pl.ds(r0, RPT)])


def _tc1_body(x_ref, wn_ref, ws_ref, b_ref, p_ref, s_ref):
    h = jnp.log(x_ref[...] + 1.0)
    p_ref[...] = jnp.dot(h, wn_ref[...], preferred_element_type=jnp.float32)
    s_ref[...] = (jnp.dot(h, ws_ref[...], preferred_element_type=jnp.float32)
                  + b_ref[...])


def _tc2_body(s1_ref, a0_ref, a1_ref, d0_ref, d1_ref, wn_ref, ws_ref, b_ref,
              p_ref, s_ref):
    deg = jnp.maximum(d0_ref[...] + d1_ref[...], 1.0)
    h = jax.nn.relu(s1_ref[...] + (a0_ref[...] + a1_ref[...]) / deg)
    nrm = jnp.sqrt(jnp.sum(h * h, axis=-1, keepdims=True))
    h = h / jnp.maximum(nrm, 1e-12)
    p_ref[...] = jnp.dot(h, wn_ref[...], preferred_element_type=jnp.float32)
    s_ref[...] = (jnp.dot(h, ws_ref[...], preferred_element_type=jnp.float32)
                  + b_ref[...])


def _tc3_body(s2_ref, a0_ref, a1_ref, d0_ref, d1_ref, o_ref):
    deg = jnp.maximum(d0_ref[...] + d1_ref[...], 1.0)
    o_ref[...] = s2_ref[...] + (a0_ref[...] + a1_ref[...]) / deg


_tc1 = pl.pallas_call(
    _tc1_body,
    out_shape=[jax.ShapeDtypeStruct((N, H), jnp.float32)] * 2,
)

_tc2 = pl.pallas_call(
    _tc2_body,
    out_shape=[jax.ShapeDtypeStruct((N, H), jnp.float32)] * 2,
)

_tc3 = pl.pallas_call(
    _tc3_body,
    out_shape=jax.ShapeDtypeStruct((N, H), jnp.float32),
)


@jax.jit
def kernel(x, edge_index, W_self1, W_neigh1, b1, W_self2, W_neigh2, b2):
    ei = edge_index.astype(jnp.int32)
    zrow = jnp.zeros((NPAD, D), jnp.float32)
    ones = jnp.ones((CH, D), jnp.float32)
    b1r = b1.reshape(1, H)
    b2r = b2.reshape(1, H)

    degp = _deg(ei, zrow, ones)
    p1, s1 = _tc1(x, W_neigh1, W_self1, b1r)
    acc1 = _seg(p1, ei, zrow)
    d0 = degp[0, :N]
    d1 = degp[1, :N]
    p2, s2 = _tc2(s1, acc1[0, :N], acc1[1, :N], d0, d1, W_neigh2, W_self2,
                  b2r)
    acc2 = _seg(p2, ei, zrow)
    out = _tc3(s2, acc2[0, :N], acc2[1, :N], d0, d1)
    return out


# speedup vs baseline: 10.2524x; 1.1298x over previous
import functools

import jax
import jax.numpy as jnp
from jax import lax
from jax.experimental import pallas as pl
from jax.experimental.pallas import tpu as pltpu
from jax.experimental.pallas import tpu_sc as plsc

N = 10000
D = 128
H = 128
E = 320000

NC = 2
NS = 16
NW = NC * NS
CH = 128
NCHUNK = E // CH
NFULL = NCHUNK // NW
NEXTRA = NCHUNK % NW
NPAD = 10112
RPT = NPAD // NS

_mesh = plsc.VectorSubcoreMesh(core_axis_name="c", subcore_axis_name="s")


def _worker_chunk_loop(do_chunk, w):
    def loop_body(k, carry):
        do_chunk(w + k * NW)
        return carry

    lax.fori_loop(0, NFULL, loop_body, 0)

    @pl.when(w < NEXTRA)
    def _():
        do_chunk(NFULL * NW + w)


@functools.partial(
    pl.kernel,
    out_type=jax.ShapeDtypeStruct((NC, NPAD, D), jnp.float32),
    mesh=_mesh,
    scratch_types=[
        pltpu.VMEM((2, CH), jnp.int32),
        pltpu.VMEM((2, CH), jnp.int32),
        pltpu.VMEM((CH, D), jnp.float32),
        pltpu.VMEM((CH, D), jnp.float32),
        pltpu.VMEM_SHARED((NPAD, D), jnp.float32),
        pltpu.SemaphoreType.DMA,
        pltpu.SemaphoreType.DMA,
    ],
)
def _seg(p_hbm, ei_hbm, zrow_hbm, acc_out,
         idx0, idx1, rows0, rows1, acc_sh, sem0, sem1):
    cid = lax.axis_index("c")
    sid = lax.axis_index("s")
    w = sid * NC + cid

    r0 = sid * RPT
    pltpu.sync_copy(zrow_hbm.at[pl.ds(r0, RPT)], acc_sh.at[pl.ds(r0, RPT)])
    plsc.subcore_barrier()

    cnt = NFULL + jnp.where(w < NEXTRA, 1, 0)

    def jbase(k):
        return jnp.where(k < NFULL, w + k * NW, NFULL * NW + w) * CH

    def fetch(k, idxb, rowsb, semb):
        pltpu.sync_copy(ei_hbm.at[:, pl.ds(jbase(k), CH)], idxb)
        pltpu.make_async_copy(p_hbm.at[idxb.at[0]], rowsb, semb).start()

    def drain(idxb, rowsb, semb):
        pltpu.make_async_copy(p_hbm.at[idxb.at[0]], rowsb, semb).wait()
        pltpu.sync_copy(rowsb, acc_sh.at[idxb.at[1]], add=True)

    fetch(0, idx0, rows0, sem0)

    def pair_body(t, carry):
        k0 = 2 * t
        fetch(k0 + 1, idx1, rows1, sem1)
        drain(idx0, rows0, sem0)

        @pl.when(k0 + 2 < cnt)
        def _():
            fetch(k0 + 2, idx0, rows0, sem0)

        drain(idx1, rows1, sem1)
        return carry

    lax.fori_loop(0, NFULL // 2, pair_body, 0)

    @pl.when(w < NEXTRA)
    def _():
        drain(idx0, rows0, sem0)

    plsc.subcore_barrier()
    pltpu.sync_copy(acc_sh.at[pl.ds(r0, RPT)],
                    acc_out.at[cid, pl.ds(r0, RPT)])


QR = 80
BK = 12800


def _tcdeg_body(dr_ref, dc_ref, o_ref):
    i = pl.program_id(0)
    q = lax.broadcasted_iota(jnp.int32, (QR, BK), 0)
    at = (q == lax.shift_right_logical(dr_ref[...], 7)).astype(jnp.bfloat16)
    r = lax.broadcasted_iota(jnp.int32, (BK, 128), 1)
    bm = (r == lax.bitwise_and(dc_ref[...], 127)).astype(jnp.bfloat16)
    blk = jnp.dot(at, bm, preferred_element_type=jnp.float32)

    @pl.when(i == 0)
    def _():
        o_ref[...] = blk

    @pl.when(i > 0)
    def _():
        o_ref[...] += blk


_tcdeg = pl.pallas_call(
    _tcdeg_body,
    grid=(E // BK,),
    in_specs=[
        pl.BlockSpec((1, BK), lambda i: (0, i)),
        pl.BlockSpec((BK, 1), lambda i: (i, 0)),
    ],
    out_specs=pl.BlockSpec((QR, 128), lambda i: (0, 0)),
    out_shape=jax.ShapeDtypeStruct((QR, 128), jnp.float32),
)


def _tc1_body(x_ref, wn_ref, ws_ref, b_ref, p_ref, s_ref):
    h = jnp.log(x_ref[...] + 1.0)
    p_ref[...] = jnp.dot(h, wn_ref[...], preferred_element_type=jnp.float32)
    s_ref[...] = (jnp.dot(h, ws_ref[...], preferred_element_type=jnp.float32)
                  + b_ref[...])


def _tc2_body(s1_ref, a0_ref, a1_ref, d_ref, wn_ref, ws_ref, b_ref,
              p_ref, s_ref):
    deg = jnp.maximum(d_ref[...], 1.0)
    h = jax.nn.relu(s1_ref[...] + (a0_ref[...] + a1_ref[...]) / deg)
    nrm = jnp.sqrt(jnp.sum(h * h, axis=-1, keepdims=True))
    h = h / jnp.maximum(nrm, 1e-12)
    p_ref[...] = jnp.dot(h, wn_ref[...], preferred_element_type=jnp.float32)
    s_ref[...] = (jnp.dot(h, ws_ref[...], preferred_element_type=jnp.float32)
                  + b_ref[...])


def _tc3_body(s2_ref, a0_ref, a1_ref, d_ref, o_ref):
    deg = jnp.maximum(d_ref[...], 1.0)
    o_ref[...] = s2_ref[...] + (a0_ref[...] + a1_ref[...]) / deg


_tc1 = pl.pallas_call(
    _tc1_body,
    out_shape=[jax.ShapeDtypeStruct((N, H), jnp.float32)] * 2,
)

_tc2 = pl.pallas_call(
    _tc2_body,
    out_shape=[jax.ShapeDtypeStruct((N, H), jnp.float32)] * 2,
)

_tc3 = pl.pallas_call(
    _tc3_body,
    out_shape=jax.ShapeDtypeStruct((N, H), jnp.float32),
)


@jax.jit
def kernel(x, edge_index, W_self1, W_neigh1, b1, W_self2, W_neigh2, b2):
    ei = edge_index.astype(jnp.int32)
    zrow = jnp.zeros((NPAD, D), jnp.float32)
    b1r = b1.reshape(1, H)
    b2r = b2.reshape(1, H)
    dst = ei[1]

    deg2d = _tcdeg(dst.reshape(1, E), dst.reshape(E, 1))
    degcol = deg2d.reshape(QR * 128)[:N, None]
    p1, s1 = _tc1(x, W_neigh1, W_self1, b1r)
    acc1 = _seg(p1, ei, zrow)
    p2, s2 = _tc2(s1, acc1[0, :N], acc1[1, :N], degcol, W_neigh2, W_self2,
                  b2r)
    acc2 = _seg(p2, ei, zrow)
    out = _tc3(s2, acc2[0, :N], acc2[1, :N], degcol)
    return out


# speedup vs baseline: 10.4497x; 1.0192x over previous
import functools

import jax
import jax.numpy as jnp
from jax import lax
from jax.experimental import pallas as pl
from jax.experimental.pallas import tpu as pltpu
from jax.experimental.pallas import tpu_sc as plsc

N = 10000
D = 128
H = 128
E = 320000

NC = 2
NS = 16
NW = NC * NS
CH = 128
NCHUNK = E // CH
NFULL = NCHUNK // NW
NEXTRA = NCHUNK % NW
NPAD = 10112
RPT = NPAD // NS

_mesh = plsc.VectorSubcoreMesh(core_axis_name="c", subcore_axis_name="s")


PL = 40


@functools.partial(
    pl.kernel,
    out_type=jax.ShapeDtypeStruct((NC, NPAD, D), jnp.float32),
    mesh=_mesh,
    scratch_types=[
        pltpu.VMEM((PL, 2, CH), jnp.int32),
        pltpu.VMEM((CH, D), jnp.float32),
        pltpu.VMEM((CH, D), jnp.float32),
        pltpu.VMEM_SHARED((NPAD, D), jnp.float32),
        pltpu.SemaphoreType.DMA,
        pltpu.SemaphoreType.DMA,
    ],
)
def _seg(p_hbm, eit_hbm, zrow_hbm, acc_out,
         idx_v, rows0, rows1, acc_sh, sem0, sem1):
    cid = lax.axis_index("c")
    sid = lax.axis_index("s")
    w = sid * NC + cid

    r0 = sid * RPT
    pltpu.sync_copy(zrow_hbm.at[pl.ds(r0, RPT)], acc_sh.at[pl.ds(r0, RPT)])

    start = NFULL * w + jnp.minimum(w, NEXTRA)
    cnt = NFULL + jnp.where(w < NEXTRA, 1, 0)
    pltpu.sync_copy(eit_hbm.at[pl.ds(start, PL)], idx_v)
    plsc.subcore_barrier()

    def fetch(r, rowsb, semb):
        pltpu.make_async_copy(p_hbm.at[idx_v.at[r, 0]], rowsb, semb).start()

    def drain(r, rowsb, semb):
        pltpu.make_async_copy(p_hbm.at[idx_v.at[r, 0]], rowsb, semb).wait()
        pltpu.sync_copy(rowsb, acc_sh.at[idx_v.at[r, 1]], add=True)

    fetch(0, rows0, sem0)

    def pair0(t, carry):
        k0 = 2 * t
        fetch(k0 + 1, rows1, sem1)
        drain(k0, rows0, sem0)

        @pl.when(k0 + 2 < PL)
        def _():
            fetch(k0 + 2, rows0, sem0)

        drain(k0 + 1, rows1, sem1)
        return carry

    lax.fori_loop(0, PL // 2, pair0, 0)

    pltpu.sync_copy(eit_hbm.at[pl.ds(start + PL, PL)], idx_v)
    rem = cnt - PL
    fetch(0, rows0, sem0)

    def pair1(t, carry):
        k0 = 2 * t
        fetch(k0 + 1, rows1, sem1)
        drain(k0, rows0, sem0)

        @pl.when(k0 + 2 < rem)
        def _():
            fetch(k0 + 2, rows0, sem0)

        drain(k0 + 1, rows1, sem1)
        return carry

    lax.fori_loop(0, (NFULL - PL) // 2, pair1, 0)

    @pl.when(w < NEXTRA)
    def _():
        drain(NFULL - PL, rows0, sem0)

    plsc.subcore_barrier()
    pltpu.sync_copy(acc_sh.at[pl.ds(r0, RPT)],
                    acc_out.at[cid, pl.ds(r0, RPT)])


QR = 80
BK = 12800


def _tcdeg_body(dr_ref, dc_ref, o_ref):
    i = pl.program_id(0)
    q = lax.broadcasted_iota(jnp.int32, (QR, BK), 0)
    at = (q == lax.shift_right_logical(dr_ref[...], 7)).astype(jnp.bfloat16)
    r = lax.broadcasted_iota(jnp.int32, (BK, 128), 1)
    bm = (r == lax.bitwise_and(dc_ref[...], 127)).astype(jnp.bfloat16)
    blk = jnp.dot(at, bm, preferred_element_type=jnp.float32)

    @pl.when(i == 0)
    def _():
        o_ref[...] = blk

    @pl.when(i > 0)
    def _():
        o_ref[...] += blk


_tcdeg = pl.pallas_call(
    _tcdeg_body,
    grid=(E // BK,),
    in_specs=[
        pl.BlockSpec((1, BK), lambda i: (0, i)),
        pl.BlockSpec((BK, 1), lambda i: (i, 0)),
    ],
    out_specs=pl.BlockSpec((QR, 128), lambda i: (0, 0)),
    out_shape=jax.ShapeDtypeStruct((QR, 128), jnp.float32),
)


def _tc1_body(x_ref, wn_ref, ws_ref, b_ref, p_ref, s_ref):
    h = jnp.log(x_ref[...] + 1.0)
    p_ref[...] = jnp.dot(h, wn_ref[...], preferred_element_type=jnp.float32)
    s_ref[...] = (jnp.dot(h, ws_ref[...], preferred_element_type=jnp.float32)
                  + b_ref[...])


def _tc2_body(s1_ref, a0_ref, a1_ref, d_ref, wn_ref, ws_ref, b_ref,
              p_ref, s_ref):
    deg = jnp.maximum(d_ref[...], 1.0)
    h = jax.nn.relu(s1_ref[...] + (a0_ref[...] + a1_ref[...]) / deg)
    nrm = jnp.sqrt(jnp.sum(h * h, axis=-1, keepdims=True))
    h = h / jnp.maximum(nrm, 1e-12)
    p_ref[...] = jnp.dot(h, wn_ref[...], preferred_element_type=jnp.float32)
    s_ref[...] = (jnp.dot(h, ws_ref[...], preferred_element_type=jnp.float32)
                  + b_ref[...])


def _tc3_body(s2_ref, a0_ref, a1_ref, d_ref, o_ref):
    deg = jnp.maximum(d_ref[...], 1.0)
    o_ref[...] = s2_ref[...] + (a0_ref[...] + a1_ref[...]) / deg


_tc1 = pl.pallas_call(
    _tc1_body,
    out_shape=[jax.ShapeDtypeStruct((N, H), jnp.float32)] * 2,
)

_tc2 = pl.pallas_call(
    _tc2_body,
    out_shape=[jax.ShapeDtypeStruct((N, H), jnp.float32)] * 2,
)

_tc3 = pl.pallas_call(
    _tc3_body,
    out_shape=jax.ShapeDtypeStruct((N, H), jnp.float32),
)


@jax.jit
def kernel(x, edge_index, W_self1, W_neigh1, b1, W_self2, W_neigh2, b2):
    ei = edge_index.astype(jnp.int32)
    eit = jnp.transpose(ei.reshape(2, NCHUNK, CH), (1, 0, 2))
    eit = jnp.pad(eit, ((0, 2 * PL), (0, 0), (0, 0)))
    zrow = jnp.zeros((NPAD, D), jnp.float32)
    b1r = b1.reshape(1, H)
    b2r = b2.reshape(1, H)
    dst = ei[1]

    deg2d = _tcdeg(dst.reshape(1, E), dst.reshape(E, 1))
    degcol = deg2d.reshape(QR * 128)[:N, None]
    p1, s1 = _tc1(x, W_neigh1, W_self1, b1r)
    acc1 = _seg(p1, eit, zrow)
    p2, s2 = _tc2(s1, acc1[0, :N], acc1[1, :N], degcol, W_neigh2, W_self2,
                  b2r)
    acc2 = _seg(p2, eit, zrow)
    out = _tc3(s2, acc2[0, :N], acc2[1, :N], degcol)
    return out


# speedup vs baseline: 10.8546x; 1.0387x over previous
import functools

import jax
import jax.numpy as jnp
from jax import lax
from jax.experimental import pallas as pl
from jax.experimental.pallas import tpu as pltpu
from jax.experimental.pallas import tpu_sc as plsc

N = 10000
D = 128
H = 128
E = 320000

NC = 2
NS = 16
NW = NC * NS
CH = 128
NCHUNK = E // CH
NFULL = NCHUNK // NW
NEXTRA = NCHUNK % NW
NPAD = 10112
RPT = NPAD // NS

_mesh = plsc.VectorSubcoreMesh(core_axis_name="c", subcore_axis_name="s")


PL = 40


@functools.partial(
    pl.kernel,
    out_type=jax.ShapeDtypeStruct((NC, NPAD, D), jnp.float32),
    mesh=_mesh,
    scratch_types=[
        pltpu.VMEM((PL, 2, CH), jnp.int32),
        pltpu.VMEM((CH, D), jnp.float32),
        pltpu.VMEM((CH, D), jnp.float32),
        pltpu.VMEM_SHARED((NPAD, D), jnp.float32),
        pltpu.SemaphoreType.DMA,
        pltpu.SemaphoreType.DMA,
    ],
)
def _seg(p_hbm, eit_hbm, zrow_hbm, acc_out,
         idx_v, rows0, rows1, acc_sh, sem0, sem1):
    cid = lax.axis_index("c")
    sid = lax.axis_index("s")
    w = sid * NC + cid

    r0 = sid * RPT
    pltpu.sync_copy(zrow_hbm.at[pl.ds(r0, RPT)], acc_sh.at[pl.ds(r0, RPT)])

    start = NFULL * w + jnp.minimum(w, NEXTRA)
    cnt = NFULL + jnp.where(w < NEXTRA, 1, 0)
    pltpu.sync_copy(eit_hbm.at[pl.ds(start, PL)], idx_v)
    plsc.subcore_barrier()

    def fetch(r, rowsb, semb):
        pltpu.make_async_copy(p_hbm.at[idx_v.at[r, 0]], rowsb, semb).start()

    def drain(r, rowsb, semb):
        pltpu.make_async_copy(p_hbm.at[idx_v.at[r, 0]], rowsb, semb).wait()
        pltpu.sync_copy(rowsb, acc_sh.at[idx_v.at[r, 1]], add=True)

    fetch(0, rows0, sem0)

    def pair0(t, carry):
        k0 = 2 * t
        fetch(k0 + 1, rows1, sem1)
        drain(k0, rows0, sem0)

        @pl.when(k0 + 2 < PL)
        def _():
            fetch(k0 + 2, rows0, sem0)

        drain(k0 + 1, rows1, sem1)
        return carry

    lax.fori_loop(0, PL // 2, pair0, 0)

    pltpu.sync_copy(eit_hbm.at[pl.ds(start + PL, PL)], idx_v)
    rem = cnt - PL
    fetch(0, rows0, sem0)

    def pair1(t, carry):
        k0 = 2 * t
        fetch(k0 + 1, rows1, sem1)
        drain(k0, rows0, sem0)

        @pl.when(k0 + 2 < rem)
        def _():
            fetch(k0 + 2, rows0, sem0)

        drain(k0 + 1, rows1, sem1)
        return carry

    lax.fori_loop(0, (NFULL - PL) // 2, pair1, 0)

    @pl.when(w < NEXTRA)
    def _():
        drain(NFULL - PL, rows0, sem0)

    plsc.subcore_barrier()
    pltpu.sync_copy(acc_sh.at[pl.ds(r0, RPT)],
                    acc_out.at[cid, pl.ds(r0, RPT)])


QR = 80
BK = 12800


def _tcdeg_body(dr_ref, dc_ref, o_ref):
    i = pl.program_id(0)
    q = lax.broadcasted_iota(jnp.int32, (QR, BK), 0)
    at = (q == lax.shift_right_logical(dr_ref[...], 7)).astype(jnp.bfloat16)
    r = lax.broadcasted_iota(jnp.int32, (BK, 128), 1)
    bm = (r == lax.bitwise_and(dc_ref[...], 127)).astype(jnp.bfloat16)
    blk = jnp.dot(at, bm, preferred_element_type=jnp.float32)

    @pl.when(i == 0)
    def _():
        o_ref[...] = blk

    @pl.when(i > 0)
    def _():
        o_ref[...] += blk


_tcdeg = pl.pallas_call(
    _tcdeg_body,
    grid=(E // BK,),
    in_specs=[
        pl.BlockSpec((1, BK), lambda i: (0, i)),
        pl.BlockSpec((BK, 1), lambda i: (i, 0)),
    ],
    out_specs=pl.BlockSpec((QR, 128), lambda i: (0, 0)),
    out_shape=jax.ShapeDtypeStruct((QR, 128), jnp.float32),
)


def _tc1_body(x_ref, wn_ref, ws_ref, b_ref, p_ref, s_ref):
    h = jnp.log(x_ref[...] + 1.0)
    p_ref[...] = jnp.dot(h, wn_ref[...], preferred_element_type=jnp.float32)
    s_ref[...] = (jnp.dot(h, ws_ref[...], preferred_element_type=jnp.float32)
                  + b_ref[...])


def _tc2_body(s1_ref, a_ref, d_ref, wn_ref, ws_ref, b_ref,
              p_ref, s_ref):
    deg = jnp.maximum(d_ref[...], 1.0)
    h = jax.nn.relu(s1_ref[...] + (a_ref[0, :N, :] + a_ref[1, :N, :]) / deg)
    nrm = jnp.sqrt(jnp.sum(h * h, axis=-1, keepdims=True))
    h = h / jnp.maximum(nrm, 1e-12)
    p_ref[...] = jnp.dot(h, wn_ref[...], preferred_element_type=jnp.float32)
    s_ref[...] = (jnp.dot(h, ws_ref[...], preferred_element_type=jnp.float32)
                  + b_ref[...])


def _tc3_body(s2_ref, a_ref, d_ref, o_ref):
    deg = jnp.maximum(d_ref[...], 1.0)
    o_ref[...] = s2_ref[...] + (a_ref[0, :N, :] + a_ref[1, :N, :]) / deg


_tc1 = pl.pallas_call(
    _tc1_body,
    out_shape=[jax.ShapeDtypeStruct((N, H), jnp.float32)] * 2,
)

_tc2 = pl.pallas_call(
    _tc2_body,
    out_shape=[jax.ShapeDtypeStruct((N, H), jnp.float32)] * 2,
)

_tc3 = pl.pallas_call(
    _tc3_body,
    out_shape=jax.ShapeDtypeStruct((N, H), jnp.float32),
)


@jax.jit
def kernel(x, edge_index, W_self1, W_neigh1, b1, W_self2, W_neigh2, b2):
    ei = edge_index.astype(jnp.int32)
    eit = jnp.transpose(ei.reshape(2, NCHUNK, CH), (1, 0, 2))
    eit = jnp.pad(eit, ((0, 2 * PL), (0, 0), (0, 0)))
    zrow = jnp.zeros((NPAD, D), jnp.float32)
    b1r = b1.reshape(1, H)
    b2r = b2.reshape(1, H)
    dst = ei[1]

    deg2d = _tcdeg(dst.reshape(1, E), dst.reshape(E, 1))
    degcol = deg2d.reshape(QR * 128)[:N, None]
    p1, s1 = _tc1(x, W_neigh1, W_self1, b1r)
    acc1 = _seg(p1, eit, zrow)
    p2, s2 = _tc2(s1, acc1, degcol, W_neigh2, W_self2, b2r)
    acc2 = _seg(p2, eit, zrow)
    out = _tc3(s2, acc2, degcol)
    return out


# speedup vs baseline: 14.0222x; 1.2918x over previous
import functools

import jax
import jax.numpy as jnp
from jax import lax
from jax.experimental import pallas as pl
from jax.experimental.pallas import tpu as pltpu
from jax.experimental.pallas import tpu_sc as plsc

N = 10000
D = 128
H = 128
E = 320000

NC = 2
NS = 16
NW = NC * NS
CH = 128
NCHUNK = E // CH
NFULL = NCHUNK // NW
NEXTRA = NCHUNK % NW
NPAD = 10112
RPT = NPAD // NS

_mesh = plsc.VectorSubcoreMesh(core_axis_name="c", subcore_axis_name="s")


PL = 40


@functools.partial(
    pl.kernel,
    out_type=jax.ShapeDtypeStruct((NC, NPAD, D), jnp.float32),
    mesh=_mesh,
    scratch_types=[
        pltpu.VMEM((PL, 2, CH), jnp.int32),
        pltpu.VMEM((CH, D), jnp.float32),
        pltpu.VMEM((CH, D), jnp.float32),
        pltpu.VMEM_SHARED((NPAD, D), jnp.float32),
        pltpu.SemaphoreType.DMA,
        pltpu.SemaphoreType.DMA,
    ],
)
def _seg(p_hbm, eit_hbm, zrow_hbm, acc_out,
         idx_v, rows0, rows1, acc_sh, sem0, sem1):
    cid = lax.axis_index("c")
    sid = lax.axis_index("s")
    w = sid * NC + cid

    r0 = sid * RPT
    pltpu.sync_copy(zrow_hbm.at[pl.ds(r0, RPT)], acc_sh.at[pl.ds(r0, RPT)])

    start = NFULL * w + jnp.minimum(w, NEXTRA)
    cnt = NFULL + jnp.where(w < NEXTRA, 1, 0)
    pltpu.sync_copy(eit_hbm.at[pl.ds(start, PL)], idx_v)
    plsc.subcore_barrier()

    def fetch(r, rowsb, semb):
        pltpu.make_async_copy(p_hbm.at[idx_v.at[r, 0]], rowsb, semb).start()

    def drain(r, rowsb, semb):
        pltpu.make_async_copy(p_hbm.at[idx_v.at[r, 0]], rowsb, semb).wait()
        pltpu.sync_copy(rowsb, acc_sh.at[idx_v.at[r, 1]], add=True)

    fetch(0, rows0, sem0)

    def pair0(t, carry):
        k0 = 2 * t
        fetch(k0 + 1, rows1, sem1)
        drain(k0, rows0, sem0)

        @pl.when(k0 + 2 < PL)
        def _():
            fetch(k0 + 2, rows0, sem0)

        drain(k0 + 1, rows1, sem1)
        return carry

    lax.fori_loop(0, PL // 2, pair0, 0)

    pltpu.sync_copy(eit_hbm.at[pl.ds(start + PL, PL)], idx_v)
    rem = cnt - PL
    fetch(0, rows0, sem0)

    def pair1(t, carry):
        k0 = 2 * t
        fetch(k0 + 1, rows1, sem1)
        drain(k0, rows0, sem0)

        @pl.when(k0 + 2 < rem)
        def _():
            fetch(k0 + 2, rows0, sem0)

        drain(k0 + 1, rows1, sem1)
        return carry

    lax.fori_loop(0, (NFULL - PL) // 2, pair1, 0)

    @pl.when(w < NEXTRA)
    def _():
        drain(NFULL - PL, rows0, sem0)

    plsc.subcore_barrier()
    pltpu.sync_copy(acc_sh.at[pl.ds(r0, RPT)],
                    acc_out.at[cid, pl.ds(r0, RPT)])


QR = 80
BK = 12800


def _tcdeg_body(dr_ref, o_ref):
    i = pl.program_id(0)
    d = dr_ref[...]
    ri = lax.broadcasted_iota(jnp.int32, (128, BK), 0)
    cm = (ri == lax.bitwise_and(d, 127)).astype(jnp.bfloat16)
    qi = lax.broadcasted_iota(jnp.int32, (QR, BK), 0)
    am = (qi == lax.shift_right_logical(d, 7)).astype(jnp.bfloat16)
    blk = lax.dot_general(cm, am, (((1,), (1,)), ((), ())),
                          preferred_element_type=jnp.float32)

    @pl.when(i == 0)
    def _():
        o_ref[...] = blk

    @pl.when(i > 0)
    def _():
        o_ref[...] += blk


_tcdeg = pl.pallas_call(
    _tcdeg_body,
    grid=(E // BK,),
    in_specs=[pl.BlockSpec((1, BK), lambda i: (0, i))],
    out_specs=pl.BlockSpec((128, QR), lambda i: (0, 0)),
    out_shape=jax.ShapeDtypeStruct((128, QR), jnp.float32),
)


def _tc1_body(x_ref, wn_ref, ws_ref, b_ref, p_ref, s_ref):
    h = jnp.log(x_ref[...] + 1.0)
    p_ref[...] = jnp.dot(h, wn_ref[...], preferred_element_type=jnp.float32)
    s_ref[...] = (jnp.dot(h, ws_ref[...], preferred_element_type=jnp.float32)
                  + b_ref[...])


def _tc2_body(s1_ref, a_ref, d_ref, wn_ref, ws_ref, b_ref,
              p_ref, s_ref):
    deg = jnp.maximum(d_ref[...], 1.0)
    h = jax.nn.relu(s1_ref[...] + (a_ref[0, :N, :] + a_ref[1, :N, :]) / deg)
    nrm = jnp.sqrt(jnp.sum(h * h, axis=-1, keepdims=True))
    h = h / jnp.maximum(nrm, 1e-12)
    p_ref[...] = jnp.dot(h, wn_ref[...], preferred_element_type=jnp.float32)
    s_ref[...] = (jnp.dot(h, ws_ref[...], preferred_element_type=jnp.float32)
                  + b_ref[...])


def _tc3_body(s2_ref, a_ref, d_ref, o_ref):
    deg = jnp.maximum(d_ref[...], 1.0)
    o_ref[...] = s2_ref[...] + (a_ref[0, :N, :] + a_ref[1, :N, :]) / deg


_tc1 = pl.pallas_call(
    _tc1_body,
    out_shape=[jax.ShapeDtypeStruct((N, H), jnp.float32)] * 2,
)

_tc2 = pl.pallas_call(
    _tc2_body,
    out_shape=[jax.ShapeDtypeStruct((N, H), jnp.float32)] * 2,
)

_tc3 = pl.pallas_call(
    _tc3_body,
    out_shape=jax.ShapeDtypeStruct((N, H), jnp.float32),
)


@jax.jit
def kernel(x, edge_index, W_self1, W_neigh1, b1, W_self2, W_neigh2, b2):
    ei = edge_index.astype(jnp.int32)
    eit = jnp.transpose(ei.reshape(2, NCHUNK, CH), (1, 0, 2))
    eit = jnp.pad(eit, ((0, 2 * PL), (0, 0), (0, 0)))
    zrow = jnp.zeros((NPAD, D), jnp.float32)
    b1r = b1.reshape(1, H)
    b2r = b2.reshape(1, H)
    dst = ei[1]

    deg2d = _tcdeg(dst.reshape(1, E))
    degcol = deg2d.T.reshape(QR * 128)[:N, None]
    p1, s1 = _tc1(x, W_neigh1, W_self1, b1r)
    acc1 = _seg(p1, eit, zrow)
    p2, s2 = _tc2(s1, acc1, degcol, W_neigh2, W_self2, b2r)
    acc2 = _seg(p2, eit, zrow)
    out = _tc3(s2, acc2, degcol)
    return out
